# baseline (speedup 1.0000x reference)
import jax
import jax.numpy as jnp
from jax.experimental import pallas as pl
from jax.experimental.pallas import tpu as pltpu

_A, _B, _C, _D = 16, 512, 64, 64
_N = _A * _B
_K = _C * _D
_ROWS = 512
_G = _N // _ROWS
_INV = 1.0 / (_A * _B * _C * _D)


def _body(x_ref, t_ref, o_ref):
    x = x_ref[...]
    g = jnp.sum(x * x, axis=1, keepdims=True) * _INV
    d = g - t_ref[0]
    o_ref[...] = jnp.broadcast_to(jnp.sum(d * d), (1, 1, 128))


@jax.jit
def kernel(x, target):
    x2 = x.reshape(_N, _K)
    t3 = target.reshape(_G, _ROWS, 1)
    partials = pl.pallas_call(
        _body,
        grid=(_G,),
        in_specs=[
            pl.BlockSpec((_ROWS, _K), lambda i: (i, 0)),
            pl.BlockSpec((1, _ROWS, 1), lambda i: (i, 0, 0)),
        ],
        out_specs=pl.BlockSpec((1, 1, 128), lambda i: (i, 0, 0)),
        out_shape=jax.ShapeDtypeStruct((_G, 1, 128), jnp.float32),
        compiler_params=pltpu.CompilerParams(
            dimension_semantics=("parallel",),
        ),
        name="style_loss_diag",
    )(x2, t3)
    return jnp.sum(partials[:, 0, 0]) / _N


# speedup vs baseline: 1.9925x; 1.9925x over previous
import jax
import jax.numpy as jnp
from jax.experimental import pallas as pl
from jax.experimental.pallas import tpu as pltpu

_A, _B, _C, _D = 16, 512, 64, 64
_N = _A * _B
_ROWS = 512
_G = _N // _ROWS
_INV = 1.0 / (_A * _B * _C * _D)


def _body(x_ref, t_ref, o_ref):
    x = x_ref[...]
    w = x * x
    s = jnp.sum(w, axis=1)
    g = jnp.sum(s, axis=1) * _INV
    d = g - t_ref[0, 0]
    o_ref[...] = jnp.broadcast_to(jnp.sum(d * d), (1, 1, 128))


@jax.jit
def kernel(x, target):
    x3 = x.reshape(_N, _C, _D)
    t3 = target.reshape(_G, 1, _ROWS)
    partials = pl.pallas_call(
        _body,
        grid=(_G,),
        in_specs=[
            pl.BlockSpec((_ROWS, _C, _D), lambda i: (i, 0, 0)),
            pl.BlockSpec((1, 1, _ROWS), lambda i: (i, 0, 0)),
        ],
        out_specs=pl.BlockSpec((1, 1, 128), lambda i: (i, 0, 0)),
        out_shape=jax.ShapeDtypeStruct((_G, 1, 128), jnp.float32),
        compiler_params=pltpu.CompilerParams(
            dimension_semantics=("parallel",),
        ),
        name="style_loss_diag",
    )(x3, t3)
    return jnp.sum(partials[:, 0, 0]) / _N


# speedup vs baseline: 8.1359x; 4.0832x over previous
import jax
import jax.numpy as jnp
from jax.experimental import pallas as pl
from jax.experimental.pallas import tpu as pltpu

_A, _B, _C, _D = 16, 512, 64, 64
_N = _A * _B
_INV = 1.0 / (_A * _B * _C * _D)


def _body(x_ref, t_ref, o_ref):
    xa = x_ref[0]
    w = xa * xa
    s = jnp.sum(w, axis=0)
    g = jnp.sum(s, axis=0) * _INV
    d = g - t_ref[0, 0]
    o_ref[...] = jnp.broadcast_to(jnp.sum(d * d), (1, 1, 128))


@jax.jit
def kernel(x, target):
    xt = jnp.transpose(x, (0, 2, 3, 1))
    t3 = target.reshape(_A, 1, _B)
    partials = pl.pallas_call(
        _body,
        grid=(_A,),
        in_specs=[
            pl.BlockSpec((1, _C, _D, _B), lambda i: (i, 0, 0, 0)),
            pl.BlockSpec((1, 1, _B), lambda i: (i, 0, 0)),
        ],
        out_specs=pl.BlockSpec((1, 1, 128), lambda i: (i, 0, 0)),
        out_shape=jax.ShapeDtypeStruct((_A, 1, 128), jnp.float32),
        compiler_params=pltpu.CompilerParams(
            dimension_semantics=("parallel",),
        ),
        name="style_loss_diag",
    )(xt, t3)
    return jnp.sum(partials[:, 0, 0]) / _N
